# baseline (speedup 1.0000x reference)
import functools

import jax
import jax.numpy as jnp
from jax import lax
from jax.experimental import pallas as pl
from jax.experimental.pallas import tpu as pltpu
from jax.experimental.pallas import tpu_sc as plsc

EMBED = 128
GB = 128

_info = plsc.get_sparse_core_info()
NC, NS = _info.num_cores, _info.num_subcores
NW = NC * NS


def _build(n_user, n_item, n_cat):
    mesh = plsc.VectorSubcoreMesh(core_axis_name="c", subcore_axis_name="s")
    out_types = tuple(
        jax.ShapeDtypeStruct((n, EMBED), jnp.float32)
        for n in (n_user, n_item, n_cat)
    )

    @functools.partial(
        pl.kernel,
        mesh=mesh,
        out_type=out_types,
        scratch_types=[
            pltpu.VMEM((GB,), jnp.int32),
            pltpu.VMEM((GB, EMBED), jnp.float32),
            pltpu.SemaphoreType.DMA,
        ],
    )
    def k(xu, xi, xc, table, ou, oi, oc, idx_v, rows_v, sem):
        wid = lax.axis_index("s") * NC + lax.axis_index("c")

        for x, o, n in ((xu, ou, n_user), (xi, oi, n_item), (xc, oc, n_cat)):
            n_full = n // GB
            tail = n - n_full * GB
            n_max = -(-n_full // NW)

            def body(i, _, x=x, o=o, n_full=n_full):
                b = wid + i * NW

                @pl.when(b < n_full)
                def _():
                    base = b * GB
                    pltpu.sync_copy(x.at[pl.ds(base, GB)], idx_v)
                    pltpu.async_copy(table.at[idx_v], rows_v, sem).wait()
                    pltpu.sync_copy(rows_v, o.at[pl.ds(base, GB)])

                return 0

            lax.fori_loop(0, n_max, body, 0)

            if tail:
                @pl.when(wid == n_full % NW)
                def _(x=x, o=o, n_full=n_full, tail=tail):
                    base = n_full * GB
                    idx_t = idx_v.at[pl.ds(0, tail)]
                    rows_t = rows_v.at[pl.ds(0, tail)]
                    pltpu.sync_copy(x.at[pl.ds(base, tail)], idx_t)
                    pltpu.async_copy(table.at[idx_t], rows_t, sem).wait()
                    pltpu.sync_copy(rows_t, o.at[pl.ds(base, tail)])

    return k


_embed3 = _build(100000, 50000, 50000)


def kernel(x_user, x_item, x_category, table):
    ou, oi, oc = _embed3(
        x_user.astype(jnp.int32),
        x_item.astype(jnp.int32),
        x_category.astype(jnp.int32),
        table,
    )
    return (ou, ou, oi, oi, oc, oc)


# speedup vs baseline: 15.3549x; 15.3549x over previous
import functools

import jax
import jax.numpy as jnp
from jax import lax
from jax.experimental import pallas as pl
from jax.experimental.pallas import tpu as pltpu
from jax.experimental.pallas import tpu_sc as plsc

EMBED = 128
GB = 128

_info = plsc.get_sparse_core_info()
NC, NS = _info.num_cores, _info.num_subcores
NW = NC * NS


def _build(n_user, n_item, n_cat):
    mesh = plsc.VectorSubcoreMesh(core_axis_name="c", subcore_axis_name="s")
    out_types = tuple(
        jax.ShapeDtypeStruct((n, EMBED), jnp.float32)
        for n in (n_user, n_item, n_cat)
    )

    @functools.partial(
        pl.kernel,
        mesh=mesh,
        out_type=out_types,
        scratch_types=[
            pltpu.VMEM((GB,), jnp.int32),
            pltpu.VMEM((GB, EMBED), jnp.float32),
            pltpu.VMEM_SHARED((3, EMBED), jnp.float32),
            pltpu.SemaphoreType.DMA,
        ],
    )
    def k(xu, xi, xc, table, ou, oi, oc, idx_v, rows_v, table_v, sem):
        wid = lax.axis_index("s") * NC + lax.axis_index("c")
        @pl.when(lax.axis_index("s") == 0)
        def _():
            pltpu.sync_copy(table, table_v)

        plsc.subcore_barrier()

        for x, o, n in ((xu, ou, n_user), (xi, oi, n_item), (xc, oc, n_cat)):
            n_full = n // GB
            tail = n - n_full * GB
            n_max = -(-n_full // NW)

            def body(i, _, x=x, o=o, n_full=n_full):
                b = wid + i * NW

                @pl.when(b < n_full)
                def _():
                    base = b * GB
                    pltpu.sync_copy(x.at[pl.ds(base, GB)], idx_v)
                    pltpu.async_copy(table_v.at[idx_v], rows_v, sem).wait()
                    pltpu.sync_copy(rows_v, o.at[pl.ds(base, GB)])

                return 0

            lax.fori_loop(0, n_max, body, 0)

            if tail:
                @pl.when(wid == n_full % NW)
                def _(x=x, o=o, n_full=n_full, tail=tail):
                    base = n_full * GB
                    idx_t = idx_v.at[pl.ds(0, tail)]
                    rows_t = rows_v.at[pl.ds(0, tail)]
                    pltpu.sync_copy(x.at[pl.ds(base, tail)], idx_t)
                    pltpu.async_copy(table_v.at[idx_t], rows_t, sem).wait()
                    pltpu.sync_copy(rows_t, o.at[pl.ds(base, tail)])

    return k


_embed3 = _build(100000, 50000, 50000)


def kernel(x_user, x_item, x_category, table):
    ou, oi, oc = _embed3(
        x_user.astype(jnp.int32),
        x_item.astype(jnp.int32),
        x_category.astype(jnp.int32),
        table,
    )
    return (ou, ou, oi, oi, oc, oc)


# speedup vs baseline: 21.0146x; 1.3686x over previous
import functools

import jax
import jax.numpy as jnp
from jax import lax
from jax.experimental import pallas as pl
from jax.experimental.pallas import tpu as pltpu
from jax.experimental.pallas import tpu_sc as plsc

EMBED = 128
GB = 128
NBUF = 6

_info = plsc.get_sparse_core_info()
NC, NS = _info.num_cores, _info.num_subcores
NW = NC * NS


def _span(n):
    s = (-(-n // NW) + 7) // 8 * 8
    assert (n - s) % 8 == 0 and s % 8 == 0
    return s


def _build(n_user, n_item, n_cat):
    ns = (n_user, n_item, n_cat)
    spans = tuple(_span(n) for n in ns)
    seg_offs = (0, spans[0], spans[0] + spans[1])
    idx_total = sum(spans)
    mesh = plsc.VectorSubcoreMesh(core_axis_name="c", subcore_axis_name="s")
    out_types = tuple(
        jax.ShapeDtypeStruct((n, EMBED), jnp.float32) for n in ns
    )

    @functools.partial(
        pl.kernel,
        mesh=mesh,
        out_type=out_types,
        scratch_types=[
            pltpu.VMEM((idx_total,), jnp.int32),
            pltpu.VMEM((NBUF, GB, EMBED), jnp.float32),
            pltpu.VMEM_SHARED((3, EMBED), jnp.float32),
        ]
        + [pltpu.SemaphoreType.DMA] * NBUF
        + [pltpu.SemaphoreType.DMA] * NBUF,
    )
    def k(xu, xi, xc, table, ou, oi, oc, idx_v, rows_v, table_s, *sems):
        gsems, ssems = sems[:NBUF], sems[NBUF:]
        wid = lax.axis_index("s") * NC + lax.axis_index("c")

        @pl.when(lax.axis_index("s") == 0)
        def _():
            pltpu.sync_copy(table, table_s)

        bases = []
        for x, n, span, soff in zip((xu, xi, xc), ns, spans, seg_offs):
            base = jnp.minimum(wid * span, n - span)
            bases.append(base)
            pltpu.sync_copy(
                x.at[pl.ds(base, span)], idx_v.at[pl.ds(soff, span)]
            )

        plsc.subcore_barrier()

        chunks = []
        for o, base, span, soff in zip((ou, oi, oc), bases, spans, seg_offs):
            n_ch = -(-span // GB)
            for c in range(n_ch):
                off = min(c * GB, span - GB)
                chunks.append((o, base, soff + off, off))

        nch = len(chunks)

        def fire_gather(ci):
            _, _, ioff, _ = chunks[ci]
            return pltpu.async_copy(
                table_s.at[idx_v.at[pl.ds(ioff, GB)]],
                rows_v.at[ci % NBUF],
                gsems[ci % NBUF],
            )

        gh = [None] * NBUF
        sh = [None] * NBUF
        for ci in range(min(NBUF, nch)):
            gh[ci] = fire_gather(ci)
        for ci in range(nch):
            b = ci % NBUF
            o, base, _, off = chunks[ci]
            gh[b].wait()
            sh[b] = pltpu.async_copy(
                rows_v.at[b], o.at[pl.ds(base + off, GB)], ssems[b]
            )
            if ci + NBUF < nch:
                sh[b].wait()
                gh[b] = fire_gather(ci + NBUF)
        for ci in range(max(0, nch - NBUF), nch):
            sh[ci % NBUF].wait()

    return k


_embed3 = _build(100000, 50000, 50000)


def kernel(x_user, x_item, x_category, table):
    ou, oi, oc = _embed3(
        x_user.astype(jnp.int32),
        x_item.astype(jnp.int32),
        x_category.astype(jnp.int32),
        table,
    )
    return (ou, ou, oi, oi, oc, oc)
